# baseline (speedup 1.0000x reference)
import functools

import jax
import jax.numpy as jnp
from jax import lax
from jax.experimental import pallas as pl
from jax.experimental.pallas import tpu as pltpu
from jax.experimental.pallas import tpu_sc as plsc

N = 10000
E = 160000
B = 64
NPAD = 10240
EPAD = 163840
EPT = EPAD // 16
CH = 128
NCHUNK = EPT // CH
RPT = NPAD // 16
DUMP = 10200
RB = 512
NRB = NPAD // RB


def _build_sc_agg(nblk: int):
    half = nblk // 2
    out_types = [jax.ShapeDtypeStruct((NPAD, 128), jnp.float32)
                 for _ in range(nblk)]
    scratch = (
        [pltpu.VMEM_SHARED((NPAD, 128), jnp.float32)]
        + [pltpu.VMEM((CH,), jnp.int32) for _ in range(4)]
        + [pltpu.VMEM((CH,), jnp.int32) for _ in range(4)]
        + [pltpu.VMEM((CH, 128), jnp.float32)] * 2
        + [pltpu.SemaphoreType.DMA] * 10
    )
    mesh = plsc.VectorSubcoreMesh(core_axis_name="c", subcore_axis_name="s")

    def body(*refs):
        pblk = refs[:nblk]
        src_hbm, dst_hbm, zrows_hbm = refs[nblk:nblk + 3]
        outs = refs[nblk + 3:2 * nblk + 3]
        sc = refs[2 * nblk + 3:]
        acc = sc[0]
        srcv = sc[1:5]
        dstv = sc[5:9]
        rows = sc[9:11]
        isem = sc[11:15]
        gsem = sc[15:17]
        gsem2 = sc[17:19]
        ssem = sc[19:21]

        HF = CH // 2

        def gather_start(p_hbm, t, b):
            pltpu.async_copy(p_hbm.at[srcv[t].at[pl.ds(0, HF)]],
                             rows[b].at[pl.ds(0, HF)], gsem[b])
            pltpu.async_copy(p_hbm.at[srcv[t].at[pl.ds(HF, HF)]],
                             rows[b].at[pl.ds(HF, HF)], gsem2[b])

        def gather_wait(p_hbm, t, b):
            pltpu.make_async_copy(p_hbm.at[srcv[t].at[pl.ds(0, HF)]],
                                  rows[b].at[pl.ds(0, HF)], gsem[b]).wait()
            pltpu.make_async_copy(p_hbm.at[srcv[t].at[pl.ds(HF, HF)]],
                                  rows[b].at[pl.ds(HF, HF)], gsem2[b]).wait()

        c = lax.axis_index("c")
        s = lax.axis_index("s")
        row0 = s * NCHUNK

        def idx_load(j, t):
            r = row0 + jnp.where(j >= NCHUNK, j - NCHUNK, j)
            pltpu.async_copy(src_hbm.at[r], srcv[t], isem[t])
            pltpu.async_copy(dst_hbm.at[r], dstv[t], isem[t])

        def idx_wait(t):
            pltpu.make_async_copy(src_hbm.at[0], srcv[t], isem[t]).wait()
            pltpu.make_async_copy(dst_hbm.at[0], dstv[t], isem[t]).wait()

        def process(pairs):
            for p_hbm, out_hbm in pairs:
                pltpu.sync_copy(zrows_hbm.at[pl.ds(s * RPT, RPT)],
                                acc.at[pl.ds(s * RPT, RPT)])
                plsc.subcore_barrier()

                idx_load(jnp.int32(0), 0)
                idx_load(jnp.int32(1), 1)
                idx_wait(0)
                gather_start(p_hbm, 0, 0)

                def step(j, k):
                    b, nb = k % 2, 1 - (k % 2)
                    gather_wait(p_hbm, k, b)
                    def wait_prev():
                        pltpu.make_async_copy(rows[nb],
                                              acc.at[dstv[(k - 1) % 4]],
                                              ssem[nb]).wait()
                    if k == 0:
                        @pl.when(j > 0)
                        def _():
                            wait_prev()
                    else:
                        wait_prev()
                    idx_load(j + 2, (k + 2) % 4)
                    idx_wait((k + 1) % 4)
                    gather_start(p_hbm, (k + 1) % 4, nb)
                    pltpu.async_copy(rows[b], acc.at[dstv[k]],
                                     ssem[b], add=True)

                def quad(i, carry):
                    for k in range(4):
                        step(4 * i + k, k)
                    return carry

                lax.fori_loop(0, NCHUNK // 4, quad, 0)
                pltpu.make_async_copy(rows[1], acc.at[dstv[3]],
                                      ssem[1]).wait()
                gather_wait(p_hbm, 0, 0)
                idx_wait(1)
                plsc.subcore_barrier()
                pltpu.sync_copy(acc.at[pl.ds(s * RPT, RPT)],
                                out_hbm.at[pl.ds(s * RPT, RPT)])
                plsc.subcore_barrier()

        @pl.when(c == 0)
        def _():
            process([(pblk[i], outs[i]) for i in range(half)])

        @pl.when(c == 1)
        def _():
            process([(pblk[i], outs[i]) for i in range(half, nblk)])

    return pl.kernel(body, out_type=tuple(out_types), mesh=mesh,
                     scratch_types=scratch)


def _build_sc_deg():
    mesh = plsc.VectorSubcoreMesh(core_axis_name="c", subcore_axis_name="s")

    def body(dst_hbm, ones_hbm, zrows_hbm, deg_out, dega, dstv, onesv):
        c = lax.axis_index("c")
        s = lax.axis_index("s")
        pltpu.sync_copy(ones_hbm, onesv)
        pltpu.sync_copy(zrows_hbm.at[pl.ds(s * RPT, RPT)],
                        dega.at[pl.ds(s * RPT, RPT)])
        plsc.subcore_barrier()

        def chunk(i, carry):
            base = s * EPT + i * CH
            pltpu.sync_copy(dst_hbm.at[pl.ds(base, CH)], dstv)
            pltpu.sync_copy(onesv, dega.at[dstv], add=True)
            return carry

        lax.fori_loop(0, NCHUNK, chunk, 0)
        plsc.subcore_barrier()

        @pl.when(c == 0)
        def _():
            pltpu.sync_copy(dega.at[pl.ds(s * RPT, RPT)],
                            deg_out.at[pl.ds(s * RPT, RPT)])

    return pl.kernel(
        body, out_type=jax.ShapeDtypeStruct((NPAD, 128), jnp.float32),
        mesh=mesh,
        scratch_types=[pltpu.VMEM_SHARED((NPAD, 128), jnp.float32),
                       pltpu.VMEM((CH,), jnp.int32),
                       pltpu.VMEM((CH, 128), jnp.float32)])


def _mm_fused(cin: int, cout: int, relu: bool):
    din, dout = cin * 128, cout * 128

    def body(*refs):
        aggr = refs[:cin]
        hr = refs[cin:2 * cin]
        recr, wl, wr, br = refs[2 * cin:2 * cin + 4]
        outs = refs[2 * cin + 4:]
        deg = recr[0][:, 0:1]
        rec = 1.0 / jnp.maximum(deg, 1.0)
        ma = jnp.zeros((RB, dout), jnp.float32)
        mh = jnp.zeros((RB, dout), jnp.float32)
        for i in range(cin):
            ma += jnp.dot(aggr[i][...], wl[pl.ds(i * 128, 128), :],
                          preferred_element_type=jnp.float32)
            mh += jnp.dot(hr[i][...], wr[pl.ds(i * 128, 128), :],
                          preferred_element_type=jnp.float32)
        z = ma * rec + mh + br[...]
        if relu:
            z = jnp.maximum(z, 0.0)
        for i in range(cout):
            outs[i][...] = z[:, i * 128:(i + 1) * 128]

    blk = pl.BlockSpec((RB, 128), lambda i: (i, 0))
    in_specs = ([blk] * (2 * cin)
                + [pl.BlockSpec((1, RB, 8), lambda i: (i, 0, 0)),
                   pl.BlockSpec((din, dout), lambda i: (0, 0)),
                   pl.BlockSpec((din, dout), lambda i: (0, 0)),
                   pl.BlockSpec((1, dout), lambda i: (0, 0))])
    return pl.pallas_call(
        body, grid=(NRB,), in_specs=in_specs,
        out_specs=[blk] * cout,
        out_shape=[jax.ShapeDtypeStruct((NPAD, 128), jnp.float32)] * cout,
    )


def _mm_plain(cin: int, cout: int):
    din, dout = cin * 128, cout * 128

    def body(*refs):
        hr = refs[:cin]
        wl = refs[cin]
        outs = refs[cin + 1:]
        m = jnp.zeros((RB, dout), jnp.float32)
        for i in range(cin):
            m += jnp.dot(hr[i][...], wl[pl.ds(i * 128, 128), :],
                         preferred_element_type=jnp.float32)
        for i in range(cout):
            outs[i][...] = m[:, i * 128:(i + 1) * 128]

    blk = pl.BlockSpec((RB, 128), lambda i: (i, 0))
    return pl.pallas_call(
        body, grid=(NRB,),
        in_specs=[blk] * cin + [pl.BlockSpec((din, dout), lambda i: (0, 0))],
        out_specs=[blk] * cout,
        out_shape=[jax.ShapeDtypeStruct((NPAD, 128), jnp.float32)] * cout,
    )


def _pool(cin_agg: int, cin_h: int):
    dout = cin_agg * 128

    def body(*refs):
        aggr = refs[:cin_agg]
        hr = refs[cin_agg:cin_agg + cin_h]
        recr, bir, wr, br = refs[cin_agg + cin_h:cin_agg + cin_h + 4]
        out = refs[cin_agg + cin_h + 4]
        pooled, cnt = refs[cin_agg + cin_h + 5:]
        i = pl.program_id(0)

        deg = recr[0][:, 0:1]
        rec = 1.0 / jnp.maximum(deg, 1.0)
        agg = jnp.concatenate([a[...] for a in aggr], axis=1)
        mh = jnp.zeros((RB, dout), jnp.float32)
        for k in range(cin_h):
            mh += jnp.dot(hr[k][...], wr[pl.ds(k * 128, 128), :],
                          preferred_element_type=jnp.float32)
        z = agg * rec + mh + br[...]

        bcol = bir[0][:, 0:1]
        oh = (bcol == lax.broadcasted_iota(jnp.int32, (1, B), 1)
              ).astype(jnp.float32)
        ps = lax.dot_general(oh, z, (((0,), (0,)), ((), ())),
                             preferred_element_type=jnp.float32)
        pc = lax.dot_general(oh, jnp.ones((RB, 8), jnp.float32),
                             (((0,), (0,)), ((), ())),
                             preferred_element_type=jnp.float32)

        @pl.when(i == 0)
        def _():
            pooled[...] = ps
            cnt[...] = pc

        @pl.when(i > 0)
        def _():
            pooled[...] += ps
            cnt[...] += pc

        @pl.when(i == NRB - 1)
        def _():
            out[...] = pooled[...] / jnp.maximum(cnt[...][:, 0:1], 1.0)

    blk = pl.BlockSpec((RB, 128), lambda i: (i, 0))
    return pl.pallas_call(
        body, grid=(NRB,),
        in_specs=[blk] * (cin_agg + cin_h)
        + [pl.BlockSpec((1, RB, 8), lambda i: (i, 0, 0)),
           pl.BlockSpec((1, RB, 8), lambda i: (i, 0, 0)),
           pl.BlockSpec((cin_h * 128, dout), lambda i: (0, 0)),
           pl.BlockSpec((1, dout), lambda i: (0, 0))],
        out_specs=pl.BlockSpec((B, dout), lambda i: (0, 0)),
        out_shape=jax.ShapeDtypeStruct((B, dout), jnp.float32),
        scratch_shapes=[pltpu.VMEM((B, dout), jnp.float32),
                        pltpu.VMEM((B, 8), jnp.float32)],
    )


_agg4 = _build_sc_agg(4)
_agg2 = _build_sc_agg(2)
_degk = _build_sc_deg()
_l0 = _mm_fused(2, 4, True)
_l1 = _mm_fused(4, 4, True)
_l2p = _mm_plain(4, 2)
_poolk = _pool(2, 4)


def _colblocks(a, nblk):
    pad = jnp.zeros((NPAD - a.shape[0], a.shape[1]), a.dtype)
    a = jnp.concatenate([a, pad], axis=0)
    return [a[:, i * 128:(i + 1) * 128] for i in range(nblk)]


def _rep8(v):
    return jnp.broadcast_to(v[:, None], (NPAD, 8)).reshape(NRB, RB, 8)


@functools.partial(jax.jit, static_argnums=())
def kernel(x, edge_index, batch_index, W_l0, b_l0, W_r0, W_l1, b_l1, W_r1,
           W_l2, b_l2, W_r2):
    src = jnp.concatenate([edge_index[0],
                           jnp.zeros((EPAD - E,), jnp.int32)])
    dst = jnp.concatenate([edge_index[1],
                           jnp.full((EPAD - E,), DUMP, jnp.int32)])
    src3 = src.reshape(16 * NCHUNK, CH)
    dst3 = dst.reshape(16 * NCHUNK, CH)
    zrows = jnp.zeros((NPAD, 128), jnp.float32)
    ones128 = jnp.ones((CH, 128), jnp.float32)

    xb = _colblocks(x, 2)
    a0, a1 = _agg2(xb[0], xb[1], src3, dst3, zrows)
    deg = _degk(dst, ones128, zrows)
    deg8 = _rep8(deg[:, 0])
    bi8 = _rep8(jnp.concatenate([batch_index,
                                 jnp.full((NPAD - N,), B, jnp.int32)]))

    h1 = _l0(a0, a1, xb[0], xb[1], deg8, W_l0, W_r0, b_l0.reshape(1, -1))
    g = _agg4(*h1, src3, dst3, zrows)
    h2 = _l1(*g, *h1, deg8, W_l1, W_r1, b_l1.reshape(1, -1))
    p2 = _l2p(*h2, W_l2)
    q = _agg2(*p2, src3, dst3, zrows)
    return _poolk(*q, *h2, deg8, bi8, W_r2, b_l2.reshape(1, -1))


# speedup vs baseline: 1.0260x; 1.0260x over previous
import functools

import jax
import jax.numpy as jnp
from jax import lax
from jax.experimental import pallas as pl
from jax.experimental.pallas import tpu as pltpu
from jax.experimental.pallas import tpu_sc as plsc

N = 10000
E = 160000
B = 64
NPAD = 10240
EPAD = 163840
EPT = EPAD // 16
CH = 128
NCHUNK = EPT // CH
RPT = NPAD // 16
DUMP = 10200
RB = 512
NRB = NPAD // RB


def _build_sc_agg(nblk: int):
    half = nblk // 2
    out_types = [jax.ShapeDtypeStruct((NPAD, 128), jnp.float32)
                 for _ in range(nblk)]
    scratch = (
        [pltpu.VMEM_SHARED((NPAD, 128), jnp.float32)]
        + [pltpu.VMEM((CH,), jnp.int32) for _ in range(4)]
        + [pltpu.VMEM((CH,), jnp.int32) for _ in range(4)]
        + [pltpu.VMEM((CH, 128), jnp.float32)] * 2
        + [pltpu.SemaphoreType.DMA] * 10
    )
    mesh = plsc.VectorSubcoreMesh(core_axis_name="c", subcore_axis_name="s")

    def body(*refs):
        pblk = refs[:nblk]
        src_hbm, dst_hbm, zrows_hbm = refs[nblk:nblk + 3]
        outs = refs[nblk + 3:2 * nblk + 3]
        sc = refs[2 * nblk + 3:]
        acc = sc[0]
        srcv = sc[1:5]
        dstv = sc[5:9]
        rows = sc[9:11]
        isem = sc[11:15]
        gsem = sc[15:17]
        gsem2 = sc[17:19]
        ssem = sc[19:21]

        HF = CH // 2

        def gather_start(p_hbm, t, b):
            pltpu.async_copy(p_hbm.at[srcv[t].at[pl.ds(0, HF)]],
                             rows[b].at[pl.ds(0, HF)], gsem[b])
            pltpu.async_copy(p_hbm.at[srcv[t].at[pl.ds(HF, HF)]],
                             rows[b].at[pl.ds(HF, HF)], gsem2[b])

        def gather_wait(p_hbm, t, b):
            pltpu.make_async_copy(p_hbm.at[srcv[t].at[pl.ds(0, HF)]],
                                  rows[b].at[pl.ds(0, HF)], gsem[b]).wait()
            pltpu.make_async_copy(p_hbm.at[srcv[t].at[pl.ds(HF, HF)]],
                                  rows[b].at[pl.ds(HF, HF)], gsem2[b]).wait()

        c = lax.axis_index("c")
        s = lax.axis_index("s")
        row0 = s * NCHUNK

        def idx_load(j, t):
            r = row0 + jnp.where(j >= NCHUNK, j - NCHUNK, j)
            pltpu.async_copy(src_hbm.at[r], srcv[t], isem[t])
            pltpu.async_copy(dst_hbm.at[r], dstv[t], isem[t])

        def idx_wait(t):
            pltpu.make_async_copy(src_hbm.at[0], srcv[t], isem[t]).wait()
            pltpu.make_async_copy(dst_hbm.at[0], dstv[t], isem[t]).wait()

        def process(pairs):
            for p_hbm, out_hbm in pairs:
                pltpu.sync_copy(zrows_hbm.at[pl.ds(s * RPT, RPT)],
                                acc.at[pl.ds(s * RPT, RPT)])
                plsc.subcore_barrier()

                idx_load(jnp.int32(0), 0)
                idx_load(jnp.int32(1), 1)
                idx_wait(0)
                gather_start(p_hbm, 0, 0)

                def step(j, k):
                    b, nb = k % 2, 1 - (k % 2)
                    gather_wait(p_hbm, k, b)
                    def wait_prev():
                        pltpu.make_async_copy(rows[nb],
                                              acc.at[dstv[(k - 1) % 4]],
                                              ssem[nb]).wait()
                    if k == 0:
                        @pl.when(j > 0)
                        def _():
                            wait_prev()
                    else:
                        wait_prev()
                    idx_load(j + 2, (k + 2) % 4)
                    idx_wait((k + 1) % 4)
                    gather_start(p_hbm, (k + 1) % 4, nb)
                    pltpu.async_copy(rows[b], acc.at[dstv[k]],
                                     ssem[b], add=True)

                def quad(i, carry):
                    for k in range(4):
                        step(4 * i + k, k)
                    return carry

                lax.fori_loop(0, NCHUNK // 4, quad, 0)
                pltpu.make_async_copy(rows[1], acc.at[dstv[3]],
                                      ssem[1]).wait()
                gather_wait(p_hbm, 0, 0)
                idx_wait(1)
                plsc.subcore_barrier()
                pltpu.sync_copy(acc.at[pl.ds(s * RPT, RPT)],
                                out_hbm.at[pl.ds(s * RPT, RPT)])
                plsc.subcore_barrier()

        @pl.when(c == 0)
        def _():
            process([(pblk[i], outs[i]) for i in range(half)])

        @pl.when(c == 1)
        def _():
            process([(pblk[i], outs[i]) for i in range(half, nblk)])

    return pl.kernel(body, out_type=tuple(out_types), mesh=mesh,
                     scratch_types=scratch)


def _build_sc_deg():
    mesh = plsc.VectorSubcoreMesh(core_axis_name="c", subcore_axis_name="s")

    def body(dst_hbm, ones_hbm, zrows_hbm, deg_out, dega, dstv, onesv):
        c = lax.axis_index("c")
        s = lax.axis_index("s")
        pltpu.sync_copy(ones_hbm, onesv)
        pltpu.sync_copy(zrows_hbm.at[pl.ds(s * RPT, RPT)],
                        dega.at[pl.ds(s * RPT, RPT)])
        plsc.subcore_barrier()

        def chunk(i, carry):
            base = s * EPT + i * CH
            pltpu.sync_copy(dst_hbm.at[pl.ds(base, CH)], dstv)
            pltpu.sync_copy(onesv, dega.at[dstv], add=True)
            return carry

        lax.fori_loop(0, NCHUNK, chunk, 0)
        plsc.subcore_barrier()

        @pl.when(c == 0)
        def _():
            pltpu.sync_copy(dega.at[pl.ds(s * RPT, RPT)],
                            deg_out.at[pl.ds(s * RPT, RPT)])

    return pl.kernel(
        body, out_type=jax.ShapeDtypeStruct((NPAD, 128), jnp.float32),
        mesh=mesh,
        scratch_types=[pltpu.VMEM_SHARED((NPAD, 128), jnp.float32),
                       pltpu.VMEM((CH,), jnp.int32),
                       pltpu.VMEM((CH, 128), jnp.float32)])


def _mm_fused(cin: int, cout: int, relu: bool, dnext: int = 0):
    din, dout = cin * 128, cout * 128

    def body(*refs):
        aggr = refs[:cin]
        hr = refs[cin:2 * cin]
        recr, wl, wr, br = refs[2 * cin:2 * cin + 4]
        k = 2 * cin + 4
        if dnext:
            wn = refs[k]
            k += 1
        outs = refs[k:]
        deg = recr[0][:, 0:1]
        rec = 1.0 / jnp.maximum(deg, 1.0)
        ma = jnp.zeros((RB, dout), jnp.float32)
        mh = jnp.zeros((RB, dout), jnp.float32)
        for i in range(cin):
            ma += jnp.dot(aggr[i][...], wl[pl.ds(i * 128, 128), :],
                          preferred_element_type=jnp.float32)
            mh += jnp.dot(hr[i][...], wr[pl.ds(i * 128, 128), :],
                          preferred_element_type=jnp.float32)
        z = ma * rec + mh + br[...]
        if relu:
            z = jnp.maximum(z, 0.0)
        for i in range(cout):
            outs[i][...] = z[:, i * 128:(i + 1) * 128]
        if dnext:
            p = jnp.dot(z, wn[...], preferred_element_type=jnp.float32)
            for i in range(dnext // 128):
                outs[cout + i][...] = p[:, i * 128:(i + 1) * 128]

    blk = pl.BlockSpec((RB, 128), lambda i: (i, 0))
    in_specs = ([blk] * (2 * cin)
                + [pl.BlockSpec((1, RB, 8), lambda i: (i, 0, 0)),
                   pl.BlockSpec((din, dout), lambda i: (0, 0)),
                   pl.BlockSpec((din, dout), lambda i: (0, 0)),
                   pl.BlockSpec((1, dout), lambda i: (0, 0))])
    nout = cout + dnext // 128
    if dnext:
        in_specs.append(pl.BlockSpec((dout, dnext), lambda i: (0, 0)))
    return pl.pallas_call(
        body, grid=(NRB,), in_specs=in_specs,
        out_specs=[blk] * nout,
        out_shape=[jax.ShapeDtypeStruct((NPAD, 128), jnp.float32)] * nout,
    )


def _mm_plain(cin: int, cout: int):
    din, dout = cin * 128, cout * 128

    def body(*refs):
        hr = refs[:cin]
        wl = refs[cin]
        outs = refs[cin + 1:]
        m = jnp.zeros((RB, dout), jnp.float32)
        for i in range(cin):
            m += jnp.dot(hr[i][...], wl[pl.ds(i * 128, 128), :],
                         preferred_element_type=jnp.float32)
        for i in range(cout):
            outs[i][...] = m[:, i * 128:(i + 1) * 128]

    blk = pl.BlockSpec((RB, 128), lambda i: (i, 0))
    return pl.pallas_call(
        body, grid=(NRB,),
        in_specs=[blk] * cin + [pl.BlockSpec((din, dout), lambda i: (0, 0))],
        out_specs=[blk] * cout,
        out_shape=[jax.ShapeDtypeStruct((NPAD, 128), jnp.float32)] * cout,
    )


def _pool(cin_agg: int, cin_h: int):
    dout = cin_agg * 128

    def body(*refs):
        aggr = refs[:cin_agg]
        hr = refs[cin_agg:cin_agg + cin_h]
        recr, bir, wr, br = refs[cin_agg + cin_h:cin_agg + cin_h + 4]
        out = refs[cin_agg + cin_h + 4]
        pooled, cnt = refs[cin_agg + cin_h + 5:]
        i = pl.program_id(0)

        deg = recr[0][:, 0:1]
        rec = 1.0 / jnp.maximum(deg, 1.0)
        agg = jnp.concatenate([a[...] for a in aggr], axis=1)
        mh = jnp.zeros((RB, dout), jnp.float32)
        for k in range(cin_h):
            mh += jnp.dot(hr[k][...], wr[pl.ds(k * 128, 128), :],
                          preferred_element_type=jnp.float32)
        z = agg * rec + mh + br[...]

        bcol = bir[0][:, 0:1]
        oh = (bcol == lax.broadcasted_iota(jnp.int32, (1, B), 1)
              ).astype(jnp.float32)
        ps = lax.dot_general(oh, z, (((0,), (0,)), ((), ())),
                             preferred_element_type=jnp.float32)
        pc = lax.dot_general(oh, jnp.ones((RB, 8), jnp.float32),
                             (((0,), (0,)), ((), ())),
                             preferred_element_type=jnp.float32)

        @pl.when(i == 0)
        def _():
            pooled[...] = ps
            cnt[...] = pc

        @pl.when(i > 0)
        def _():
            pooled[...] += ps
            cnt[...] += pc

        @pl.when(i == NRB - 1)
        def _():
            out[...] = pooled[...] / jnp.maximum(cnt[...][:, 0:1], 1.0)

    blk = pl.BlockSpec((RB, 128), lambda i: (i, 0))
    return pl.pallas_call(
        body, grid=(NRB,),
        in_specs=[blk] * (cin_agg + cin_h)
        + [pl.BlockSpec((1, RB, 8), lambda i: (i, 0, 0)),
           pl.BlockSpec((1, RB, 8), lambda i: (i, 0, 0)),
           pl.BlockSpec((cin_h * 128, dout), lambda i: (0, 0)),
           pl.BlockSpec((1, dout), lambda i: (0, 0))],
        out_specs=pl.BlockSpec((B, dout), lambda i: (0, 0)),
        out_shape=jax.ShapeDtypeStruct((B, dout), jnp.float32),
        scratch_shapes=[pltpu.VMEM((B, dout), jnp.float32),
                        pltpu.VMEM((B, 8), jnp.float32)],
    )


_agg4 = _build_sc_agg(4)
_agg2 = _build_sc_agg(2)
_degk = _build_sc_deg()
_l0 = _mm_fused(2, 4, True)
_l1 = _mm_fused(4, 4, True, dnext=256)
_poolk = _pool(2, 4)


def _colblocks(a, nblk):
    pad = jnp.zeros((NPAD - a.shape[0], a.shape[1]), a.dtype)
    a = jnp.concatenate([a, pad], axis=0)
    return [a[:, i * 128:(i + 1) * 128] for i in range(nblk)]


def _rep8(v):
    return jnp.broadcast_to(v[:, None], (NPAD, 8)).reshape(NRB, RB, 8)


@functools.partial(jax.jit, static_argnums=())
def kernel(x, edge_index, batch_index, W_l0, b_l0, W_r0, W_l1, b_l1, W_r1,
           W_l2, b_l2, W_r2):
    src = jnp.concatenate([edge_index[0],
                           jnp.zeros((EPAD - E,), jnp.int32)])
    dst = jnp.concatenate([edge_index[1],
                           jnp.full((EPAD - E,), DUMP, jnp.int32)])
    src3 = src.reshape(16 * NCHUNK, CH)
    dst3 = dst.reshape(16 * NCHUNK, CH)
    zrows = jnp.zeros((NPAD, 128), jnp.float32)
    ones128 = jnp.ones((CH, 128), jnp.float32)

    xb = _colblocks(x, 2)
    a0, a1 = _agg2(xb[0], xb[1], src3, dst3, zrows)
    deg = _degk(dst, ones128, zrows)
    deg8 = _rep8(deg[:, 0])
    bi8 = _rep8(jnp.concatenate([batch_index,
                                 jnp.full((NPAD - N,), B, jnp.int32)]))

    h1 = _l0(a0, a1, xb[0], xb[1], deg8, W_l0, W_r0, b_l0.reshape(1, -1))
    g = _agg4(*h1, src3, dst3, zrows)
    out1 = _l1(*g, *h1, deg8, W_l1, W_r1, b_l1.reshape(1, -1), W_l2)
    h2, p2 = out1[:4], out1[4:]
    q = _agg2(*p2, src3, dst3, zrows)
    return _poolk(*q, *h2, deg8, bi8, W_r2, b_l2.reshape(1, -1))
